# baseline (speedup 1.0000x reference)
import functools

import jax
import jax.numpy as jnp
from jax import lax
from jax.experimental import pallas as pl
from jax.experimental.pallas import tpu as pltpu
from jax.experimental.pallas import tpu_sc as plsc

VOCAB = 100000
DESC = 768
PROJ = 128
BATCH = 16384


@functools.cache
def _make_sc_gather(B, D):
    NC, NS = 2, 16
    NW = NC * NS
    b_per_w = B // NW
    C = 64
    n_chunks = b_per_w // C
    mesh = plsc.VectorSubcoreMesh(core_axis_name="c", subcore_axis_name="s")

    @functools.partial(
        pl.kernel,
        mesh=mesh,
        out_type=jax.ShapeDtypeStruct((B, D), jnp.float32),
        scratch_types=[
            pltpu.VMEM((b_per_w,), jnp.int32),
            pltpu.VMEM((C, D), jnp.float32),
            pltpu.VMEM((C, D), jnp.float32),
            pltpu.SemaphoreType.DMA,
            pltpu.SemaphoreType.DMA,
        ],
    )
    def gather(idx_hbm, table_hbm, out_hbm, idx_v, f0, f1, g0, g1):
        wid = lax.axis_index("s") * NC + lax.axis_index("c")
        base = wid * b_per_w
        pltpu.sync_copy(idx_hbm.at[pl.ds(base, b_per_w)], idx_v)

        pltpu.async_copy(table_hbm.at[idx_v.at[pl.ds(0, C)]], f0, g0)

        def chunk(i, carry):
            def do(fb, gs, fb_n, gs_n):
                @pl.when(i + 1 < n_chunks)
                def _():
                    off = pl.multiple_of((i + 1) * C, 8)
                    pltpu.async_copy(
                        table_hbm.at[idx_v.at[pl.ds(off, C)]], fb_n, gs_n)
                pltpu.make_async_copy(
                    table_hbm.at[idx_v.at[pl.ds(0, C)]], fb, gs).wait()
                pltpu.sync_copy(fb, out_hbm.at[pl.ds(base + i * C, C)])

            @pl.when(lax.rem(i, 2) == 0)
            def _():
                do(f0, g0, f1, g1)

            @pl.when(lax.rem(i, 2) == 1)
            def _():
                do(f1, g1, f0, g0)

            return carry

        lax.fori_loop(0, n_chunks, chunk, 0)

    return gather


_BM = 2048


def _proj_body(vals_ref, emb_ref, wproj_ref, wvalt_ref, bval_ref, bproj_ref, out_ref):
    wp1 = wproj_ref[:, :DESC]
    wp2 = wproj_ref[:, DESC:]
    c1 = lax.dot_general(wvalt_ref[...], wp1, (((1,), (1,)), ((), ())),
                         preferred_element_type=jnp.float32)
    c0 = lax.dot_general(bval_ref[...], wp1, (((1,), (1,)), ((), ())),
                         preferred_element_type=jnp.float32) + bproj_ref[...]
    emb_term = lax.dot_general(emb_ref[...], wp2, (((1,), (1,)), ((), ())),
                               preferred_element_type=jnp.float32)
    val_term = lax.dot_general(vals_ref[...], c1, (((1,), (0,)), ((), ())),
                               preferred_element_type=jnp.float32)
    out_ref[...] = emb_term + val_term + c0


def _proj(values, emb, W_proj, W_val_t, b_val2, b_proj2):
    n = values.shape[0]
    grid = (n // _BM,)
    return pl.pallas_call(
        _proj_body,
        grid=grid,
        in_specs=[
            pl.BlockSpec((_BM, 1), lambda i: (i, 0)),
            pl.BlockSpec((_BM, DESC), lambda i: (i, 0)),
            pl.BlockSpec((PROJ, 2 * DESC), lambda i: (0, 0)),
            pl.BlockSpec((1, DESC), lambda i: (0, 0)),
            pl.BlockSpec((1, DESC), lambda i: (0, 0)),
            pl.BlockSpec((1, PROJ), lambda i: (0, 0)),
        ],
        out_specs=pl.BlockSpec((_BM, PROJ), lambda i: (i, 0)),
        out_shape=jax.ShapeDtypeStruct((n, PROJ), jnp.float32),
    )(values, emb, W_proj, W_val_t, b_val2, b_proj2)


_NSLICE = 2


def kernel(description_idx, values, embedded_descriptions, W_val, b_val, W_proj, b_proj):
    idx = description_idx.astype(jnp.int32)
    wvalt = W_val.reshape(1, DESC)
    bval2 = b_val.reshape(1, DESC)
    bproj2 = b_proj.reshape(1, PROJ)
    S = BATCH // _NSLICE
    gather = _make_sc_gather(S, DESC)
    embs = [gather(idx[s * S:(s + 1) * S], embedded_descriptions)
            for s in range(_NSLICE)]
    outs = [_proj(values[s * S:(s + 1) * S], embs[s], W_proj, wvalt, bval2,
                  bproj2) for s in range(_NSLICE)]
    return jnp.concatenate(outs, axis=0)


# speedup vs baseline: 1.1830x; 1.1830x over previous
import functools

import jax
import jax.numpy as jnp
from jax import lax
from jax.experimental import pallas as pl
from jax.experimental.pallas import tpu as pltpu
from jax.experimental.pallas import tpu_sc as plsc

VOCAB = 100000
DESC = 768
PROJ = 128
BATCH = 16384


@functools.cache
def _make_sc_gather(B, D):
    NC, NS = 2, 16
    NW = NC * NS
    b_per_w = B // NW
    C = 64
    n_chunks = b_per_w // C
    mesh = plsc.VectorSubcoreMesh(core_axis_name="c", subcore_axis_name="s")

    @functools.partial(
        pl.kernel,
        mesh=mesh,
        out_type=jax.ShapeDtypeStruct((B, D), jnp.float32),
        scratch_types=[
            pltpu.VMEM((b_per_w,), jnp.int32),
            pltpu.VMEM((C, D), jnp.float32),
            pltpu.VMEM((C, D), jnp.float32),
            pltpu.SemaphoreType.DMA,
            pltpu.SemaphoreType.DMA,
        ],
    )
    def gather(idx_hbm, table_hbm, out_hbm, idx_v, f0, f1, g0, g1):
        wid = lax.axis_index("s") * NC + lax.axis_index("c")
        base = wid * b_per_w
        pltpu.sync_copy(idx_hbm.at[pl.ds(base, b_per_w)], idx_v)

        pltpu.async_copy(table_hbm.at[idx_v.at[pl.ds(0, C)]], f0, g0)

        def chunk(i, carry):
            def do(fb, gs, fb_n, gs_n):
                @pl.when(i + 1 < n_chunks)
                def _():
                    off = pl.multiple_of((i + 1) * C, 8)
                    pltpu.async_copy(
                        table_hbm.at[idx_v.at[pl.ds(off, C)]], fb_n, gs_n)
                pltpu.make_async_copy(
                    table_hbm.at[idx_v.at[pl.ds(0, C)]], fb, gs).wait()
                pltpu.sync_copy(fb, out_hbm.at[pl.ds(base + i * C, C)])

            @pl.when(lax.rem(i, 2) == 0)
            def _():
                do(f0, g0, f1, g1)

            @pl.when(lax.rem(i, 2) == 1)
            def _():
                do(f1, g1, f0, g0)

            return carry

        lax.fori_loop(0, n_chunks, chunk, 0)

    return gather


_BM = 4096


def _proj_body(vals_ref, emb_ref, wproj_ref, wvalt_ref, bval_ref, bproj_ref, out_ref):
    wp1 = wproj_ref[:, :DESC]
    wp2 = wproj_ref[:, DESC:]
    c1 = lax.dot_general(wvalt_ref[...], wp1, (((1,), (1,)), ((), ())),
                         preferred_element_type=jnp.float32)
    c0 = lax.dot_general(bval_ref[...], wp1, (((1,), (1,)), ((), ())),
                         preferred_element_type=jnp.float32) + bproj_ref[...]
    emb_term = lax.dot_general(emb_ref[...], wp2, (((1,), (1,)), ((), ())),
                               preferred_element_type=jnp.float32)
    val_term = lax.dot_general(vals_ref[...], c1, (((0,), (0,)), ((), ())),
                               preferred_element_type=jnp.float32)
    out_ref[...] = emb_term + val_term + c0


def _proj(values_row, emb, W_proj, W_val_t, b_val2, b_proj2):
    n = emb.shape[0]
    grid = (n // _BM,)
    return pl.pallas_call(
        _proj_body,
        grid=grid,
        in_specs=[
            pl.BlockSpec((1, _BM), lambda i: (0, i)),
            pl.BlockSpec((_BM, DESC), lambda i: (i, 0)),
            pl.BlockSpec((PROJ, 2 * DESC), lambda i: (0, 0)),
            pl.BlockSpec((1, DESC), lambda i: (0, 0)),
            pl.BlockSpec((1, DESC), lambda i: (0, 0)),
            pl.BlockSpec((1, PROJ), lambda i: (0, 0)),
        ],
        out_specs=pl.BlockSpec((_BM, PROJ), lambda i: (i, 0)),
        out_shape=jax.ShapeDtypeStruct((n, PROJ), jnp.float32),
    )(values_row, emb, W_proj, W_val_t, b_val2, b_proj2)


_NSLICE = 1


def kernel(description_idx, values, embedded_descriptions, W_val, b_val, W_proj, b_proj):
    idx = description_idx.astype(jnp.int32)
    wvalt = W_val.reshape(1, DESC)
    bval2 = b_val.reshape(1, DESC)
    bproj2 = b_proj.reshape(1, PROJ)
    values_row = values.reshape(1, BATCH)
    if _NSLICE == 1:
        emb = _make_sc_gather(BATCH, DESC)(idx, embedded_descriptions)
        return _proj(values_row, emb, W_proj, wvalt, bval2, bproj2)
    S = BATCH // _NSLICE
    gather = _make_sc_gather(S, DESC)
    embs = [gather(idx[s * S:(s + 1) * S], embedded_descriptions)
            for s in range(_NSLICE)]
    outs = [_proj(values_row[:, s * S:(s + 1) * S], embs[s], W_proj, wvalt,
                  bval2, bproj2) for s in range(_NSLICE)]
    return jnp.concatenate(outs, axis=0)


# speedup vs baseline: 1.1853x; 1.0020x over previous
import functools

import jax
import jax.numpy as jnp
from jax import lax
from jax.experimental import pallas as pl
from jax.experimental.pallas import tpu as pltpu
from jax.experimental.pallas import tpu_sc as plsc

VOCAB = 100000
DESC = 768
PROJ = 128
BATCH = 16384


@functools.cache
def _make_sc_gather(B, D):
    NC, NS = 2, 16
    NW = NC * NS
    b_per_w = B // NW
    C = 64
    n_chunks = b_per_w // C
    mesh = plsc.VectorSubcoreMesh(core_axis_name="c", subcore_axis_name="s")

    @functools.partial(
        pl.kernel,
        mesh=mesh,
        compiler_params=pltpu.CompilerParams(skip_device_barrier=True),
        out_type=jax.ShapeDtypeStruct((B, D), jnp.float32),
        scratch_types=[
            pltpu.VMEM((b_per_w,), jnp.int32),
            pltpu.VMEM((C, D), jnp.float32),
            pltpu.VMEM((C, D), jnp.float32),
            pltpu.SemaphoreType.DMA,
            pltpu.SemaphoreType.DMA,
        ],
    )
    def gather(idx_hbm, table_hbm, out_hbm, idx_v, f0, f1, g0, g1):
        wid = lax.axis_index("s") * NC + lax.axis_index("c")
        base = wid * b_per_w
        pltpu.sync_copy(idx_hbm.at[pl.ds(base, b_per_w)], idx_v)

        pltpu.async_copy(table_hbm.at[idx_v.at[pl.ds(0, C)]], f0, g0)

        def chunk(i, carry):
            def do(fb, gs, fb_n, gs_n):
                @pl.when(i + 1 < n_chunks)
                def _():
                    off = pl.multiple_of((i + 1) * C, 8)
                    pltpu.async_copy(
                        table_hbm.at[idx_v.at[pl.ds(off, C)]], fb_n, gs_n)
                pltpu.make_async_copy(
                    table_hbm.at[idx_v.at[pl.ds(0, C)]], fb, gs).wait()
                pltpu.sync_copy(fb, out_hbm.at[pl.ds(base + i * C, C)])

            @pl.when(lax.rem(i, 2) == 0)
            def _():
                do(f0, g0, f1, g1)

            @pl.when(lax.rem(i, 2) == 1)
            def _():
                do(f1, g1, f0, g0)

            return carry

        lax.fori_loop(0, n_chunks, chunk, 0)

    return gather


_BM = 4096


def _proj_body(vals_ref, emb_ref, wproj_ref, wvalt_ref, bval_ref, bproj_ref, out_ref):
    wp1 = wproj_ref[:, :DESC]
    wp2 = wproj_ref[:, DESC:]
    c1 = lax.dot_general(wvalt_ref[...], wp1, (((1,), (1,)), ((), ())),
                         preferred_element_type=jnp.float32)
    c0 = lax.dot_general(bval_ref[...], wp1, (((1,), (1,)), ((), ())),
                         preferred_element_type=jnp.float32) + bproj_ref[...]
    emb_term = lax.dot_general(emb_ref[...], wp2, (((1,), (1,)), ((), ())),
                               preferred_element_type=jnp.float32)
    val_term = lax.dot_general(vals_ref[...], c1, (((0,), (0,)), ((), ())),
                               preferred_element_type=jnp.float32)
    out_ref[...] = emb_term + val_term + c0


def _proj(values_row, emb, W_proj, W_val_t, b_val2, b_proj2):
    n = emb.shape[0]
    grid = (n // _BM,)
    return pl.pallas_call(
        _proj_body,
        grid=grid,
        in_specs=[
            pl.BlockSpec((1, _BM), lambda i: (0, i)),
            pl.BlockSpec((_BM, DESC), lambda i: (i, 0)),
            pl.BlockSpec((PROJ, 2 * DESC), lambda i: (0, 0)),
            pl.BlockSpec((1, DESC), lambda i: (0, 0)),
            pl.BlockSpec((1, DESC), lambda i: (0, 0)),
            pl.BlockSpec((1, PROJ), lambda i: (0, 0)),
        ],
        out_specs=pl.BlockSpec((_BM, PROJ), lambda i: (i, 0)),
        out_shape=jax.ShapeDtypeStruct((n, PROJ), jnp.float32),
        compiler_params=pltpu.CompilerParams(skip_device_barrier=True),
    )(values_row, emb, W_proj, W_val_t, b_val2, b_proj2)


_NSLICE = 1


def kernel(description_idx, values, embedded_descriptions, W_val, b_val, W_proj, b_proj):
    idx = description_idx.astype(jnp.int32)
    wvalt = W_val.reshape(1, DESC)
    bval2 = b_val.reshape(1, DESC)
    bproj2 = b_proj.reshape(1, PROJ)
    values_row = values.reshape(1, BATCH)
    if _NSLICE == 1:
        emb = _make_sc_gather(BATCH, DESC)(idx, embedded_descriptions)
        return _proj(values_row, emb, W_proj, wvalt, bval2, bproj2)
    S = BATCH // _NSLICE
    gather = _make_sc_gather(S, DESC)
    embs = [gather(idx[s * S:(s + 1) * S], embedded_descriptions)
            for s in range(_NSLICE)]
    outs = [_proj(values_row[:, s * S:(s + 1) * S], embs[s], W_proj, wvalt,
                  bval2, bproj2) for s in range(_NSLICE)]
    return jnp.concatenate(outs, axis=0)


# speedup vs baseline: 1.1932x; 1.0067x over previous
import functools

import jax
import jax.numpy as jnp
from jax import lax
from jax.experimental import pallas as pl
from jax.experimental.pallas import tpu as pltpu
from jax.experimental.pallas import tpu_sc as plsc

VOCAB = 100000
DESC = 768
PROJ = 128
BATCH = 16384


@functools.cache
def _make_sc_gather(B, D):
    NC, NS = 2, 16
    NW = NC * NS
    b_per_w = B // NW
    C = 32
    NB = 4
    n_chunks = b_per_w // C
    mesh = plsc.VectorSubcoreMesh(core_axis_name="c", subcore_axis_name="s")

    @functools.partial(
        pl.kernel,
        mesh=mesh,
        compiler_params=pltpu.CompilerParams(skip_device_barrier=True),
        out_type=jax.ShapeDtypeStruct((B, D), jnp.float32),
        scratch_types=[
            pltpu.VMEM((b_per_w,), jnp.int32),
            [pltpu.VMEM((C, D), jnp.float32) for _ in range(NB)],
            [pltpu.SemaphoreType.DMA for _ in range(NB)],
            [pltpu.SemaphoreType.DMA for _ in range(NB)],
        ],
    )
    def gather(idx_hbm, table_hbm, out_hbm, idx_v, bufs, gsems, osems):
        wid = lax.axis_index("s") * NC + lax.axis_index("c")
        base = wid * b_per_w
        pltpu.sync_copy(idx_hbm.at[pl.ds(base, b_per_w)], idx_v)

        def issue_gather(c, buf, gs):
            off = pl.multiple_of(c * C, 8)
            pltpu.async_copy(table_hbm.at[idx_v.at[pl.ds(off, C)]], buf, gs)

        for k in range(NB - 1):
            issue_gather(k, bufs[k], gsems[k])

        def chunk(i, carry):
            def do(s):
                s_next = (s + NB - 1) % NB
                @pl.when(i >= 1)
                def _():
                    pltpu.make_async_copy(
                        bufs[s_next], out_hbm.at[pl.ds(base, C)],
                        osems[s_next]).wait()
                @pl.when(i + NB - 1 < n_chunks)
                def _():
                    issue_gather(i + NB - 1, bufs[s_next], gsems[s_next])
                pltpu.make_async_copy(
                    table_hbm.at[idx_v.at[pl.ds(0, C)]], bufs[s],
                    gsems[s]).wait()
                pltpu.async_copy(
                    bufs[s], out_hbm.at[pl.ds(base + i * C, C)], osems[s])

            for s in range(NB):
                @pl.when(lax.rem(i, NB) == s)
                def _(s=s):
                    do(s)

            return carry

        lax.fori_loop(0, n_chunks, chunk, 0)
        s_last = (n_chunks - 1) % NB
        pltpu.make_async_copy(
            bufs[s_last], out_hbm.at[pl.ds(base, C)], osems[s_last]).wait()

    return gather


_BM = 4096


def _proj_body(vals_ref, emb_ref, wproj_ref, wvalt_ref, bval_ref, bproj_ref, out_ref):
    wp1 = wproj_ref[:, :DESC]
    wp2 = wproj_ref[:, DESC:]
    c1 = lax.dot_general(wvalt_ref[...], wp1, (((1,), (1,)), ((), ())),
                         preferred_element_type=jnp.float32)
    c0 = lax.dot_general(bval_ref[...], wp1, (((1,), (1,)), ((), ())),
                         preferred_element_type=jnp.float32) + bproj_ref[...]
    emb_term = lax.dot_general(emb_ref[...], wp2, (((1,), (1,)), ((), ())),
                               preferred_element_type=jnp.float32)
    val_term = lax.dot_general(vals_ref[...], c1, (((0,), (0,)), ((), ())),
                               preferred_element_type=jnp.float32)
    out_ref[...] = emb_term + val_term + c0


def _proj(values_row, emb, W_proj, W_val_t, b_val2, b_proj2):
    n = emb.shape[0]
    grid = (n // _BM,)
    return pl.pallas_call(
        _proj_body,
        grid=grid,
        in_specs=[
            pl.BlockSpec((1, _BM), lambda i: (0, i)),
            pl.BlockSpec((_BM, DESC), lambda i: (i, 0)),
            pl.BlockSpec((PROJ, 2 * DESC), lambda i: (0, 0)),
            pl.BlockSpec((1, DESC), lambda i: (0, 0)),
            pl.BlockSpec((1, DESC), lambda i: (0, 0)),
            pl.BlockSpec((1, PROJ), lambda i: (0, 0)),
        ],
        out_specs=pl.BlockSpec((_BM, PROJ), lambda i: (i, 0)),
        out_shape=jax.ShapeDtypeStruct((n, PROJ), jnp.float32),
        compiler_params=pltpu.CompilerParams(skip_device_barrier=True),
    )(values_row, emb, W_proj, W_val_t, b_val2, b_proj2)


_NSLICE = 1


def kernel(description_idx, values, embedded_descriptions, W_val, b_val, W_proj, b_proj):
    idx = description_idx.astype(jnp.int32)
    wvalt = W_val.reshape(1, DESC)
    bval2 = b_val.reshape(1, DESC)
    bproj2 = b_proj.reshape(1, PROJ)
    values_row = values.reshape(1, BATCH)
    if _NSLICE == 1:
        emb = _make_sc_gather(BATCH, DESC)(idx, embedded_descriptions)
        return _proj(values_row, emb, W_proj, wvalt, bval2, bproj2)
    S = BATCH // _NSLICE
    gather = _make_sc_gather(S, DESC)
    embs = [gather(idx[s * S:(s + 1) * S], embedded_descriptions)
            for s in range(_NSLICE)]
    outs = [_proj(values_row[:, s * S:(s + 1) * S], embs[s], W_proj, wvalt,
                  bval2, bproj2) for s in range(_NSLICE)]
    return jnp.concatenate(outs, axis=0)


# speedup vs baseline: 1.1933x; 1.0001x over previous
import functools

import jax
import jax.numpy as jnp
from jax import lax
from jax.experimental import pallas as pl
from jax.experimental.pallas import tpu as pltpu
from jax.experimental.pallas import tpu_sc as plsc

VOCAB = 100000
DESC = 768
PROJ = 128
BATCH = 16384


@functools.cache
def _make_sc_gather(B, D):
    NC, NS = 2, 16
    NW = NC * NS
    b_per_w = B // NW
    C = 32
    NB = 5
    n_chunks = b_per_w // C
    mesh = plsc.VectorSubcoreMesh(core_axis_name="c", subcore_axis_name="s")

    @functools.partial(
        pl.kernel,
        mesh=mesh,
        compiler_params=pltpu.CompilerParams(skip_device_barrier=True),
        out_type=jax.ShapeDtypeStruct((B, D), jnp.float32),
        scratch_types=[
            pltpu.VMEM((b_per_w,), jnp.int32),
            [pltpu.VMEM((C, D), jnp.float32) for _ in range(NB)],
            [pltpu.SemaphoreType.DMA for _ in range(NB)],
            [pltpu.SemaphoreType.DMA for _ in range(NB)],
        ],
    )
    def gather(idx_hbm, table_hbm, out_hbm, idx_v, bufs, gsems, osems):
        wid = lax.axis_index("s") * NC + lax.axis_index("c")
        base = wid * b_per_w
        pltpu.sync_copy(idx_hbm.at[pl.ds(base, b_per_w)], idx_v)

        def issue_gather(c, buf, gs):
            off = pl.multiple_of(c * C, 8)
            pltpu.async_copy(table_hbm.at[idx_v.at[pl.ds(off, C)]], buf, gs)

        for k in range(NB - 1):
            issue_gather(k, bufs[k], gsems[k])

        def chunk(i, carry):
            def do(s):
                s_next = (s + NB - 1) % NB
                @pl.when(i >= 1)
                def _():
                    pltpu.make_async_copy(
                        bufs[s_next], out_hbm.at[pl.ds(base, C)],
                        osems[s_next]).wait()
                @pl.when(i + NB - 1 < n_chunks)
                def _():
                    issue_gather(i + NB - 1, bufs[s_next], gsems[s_next])
                pltpu.make_async_copy(
                    table_hbm.at[idx_v.at[pl.ds(0, C)]], bufs[s],
                    gsems[s]).wait()
                pltpu.async_copy(
                    bufs[s], out_hbm.at[pl.ds(base + i * C, C)], osems[s])

            for s in range(NB):
                @pl.when(lax.rem(i, NB) == s)
                def _(s=s):
                    do(s)

            return carry

        lax.fori_loop(0, n_chunks, chunk, 0)
        s_last = (n_chunks - 1) % NB
        pltpu.make_async_copy(
            bufs[s_last], out_hbm.at[pl.ds(base, C)], osems[s_last]).wait()

    return gather


_BM = 4096


def _proj_body(vals_ref, emb_ref, wproj_ref, wvalt_ref, bval_ref, bproj_ref, out_ref):
    wp1 = wproj_ref[:, :DESC]
    wp2 = wproj_ref[:, DESC:]
    c1 = lax.dot_general(wvalt_ref[...], wp1, (((1,), (1,)), ((), ())),
                         preferred_element_type=jnp.float32)
    c0 = lax.dot_general(bval_ref[...], wp1, (((1,), (1,)), ((), ())),
                         preferred_element_type=jnp.float32) + bproj_ref[...]
    emb_term = lax.dot_general(emb_ref[...], wp2, (((1,), (1,)), ((), ())),
                               preferred_element_type=jnp.float32)
    val_term = lax.dot_general(vals_ref[...], c1, (((0,), (0,)), ((), ())),
                               preferred_element_type=jnp.float32)
    out_ref[...] = emb_term + val_term + c0


def _proj(values_row, emb, W_proj, W_val_t, b_val2, b_proj2):
    n = emb.shape[0]
    grid = (n // _BM,)
    return pl.pallas_call(
        _proj_body,
        grid=grid,
        in_specs=[
            pl.BlockSpec((1, _BM), lambda i: (0, i)),
            pl.BlockSpec((_BM, DESC), lambda i: (i, 0)),
            pl.BlockSpec((PROJ, 2 * DESC), lambda i: (0, 0)),
            pl.BlockSpec((1, DESC), lambda i: (0, 0)),
            pl.BlockSpec((1, DESC), lambda i: (0, 0)),
            pl.BlockSpec((1, PROJ), lambda i: (0, 0)),
        ],
        out_specs=pl.BlockSpec((_BM, PROJ), lambda i: (i, 0)),
        out_shape=jax.ShapeDtypeStruct((n, PROJ), jnp.float32),
        compiler_params=pltpu.CompilerParams(skip_device_barrier=True),
    )(values_row, emb, W_proj, W_val_t, b_val2, b_proj2)


_NSLICE = 1


def kernel(description_idx, values, embedded_descriptions, W_val, b_val, W_proj, b_proj):
    idx = description_idx.astype(jnp.int32)
    wvalt = W_val.reshape(1, DESC)
    bval2 = b_val.reshape(1, DESC)
    bproj2 = b_proj.reshape(1, PROJ)
    values_row = values.reshape(1, BATCH)
    if _NSLICE == 1:
        emb = _make_sc_gather(BATCH, DESC)(idx, embedded_descriptions)
        return _proj(values_row, emb, W_proj, wvalt, bval2, bproj2)
    S = BATCH // _NSLICE
    gather = _make_sc_gather(S, DESC)
    embs = [gather(idx[s * S:(s + 1) * S], embedded_descriptions)
            for s in range(_NSLICE)]
    outs = [_proj(values_row[:, s * S:(s + 1) * S], embs[s], W_proj, wvalt,
                  bval2, bproj2) for s in range(_NSLICE)]
    return jnp.concatenate(outs, axis=0)
